# baseline (speedup 1.0000x reference)
import functools

import jax
import jax.numpy as jnp
from jax import lax
from jax.experimental import pallas as pl
from jax.experimental.pallas import tpu as pltpu
from jax.experimental.pallas import tpu_sc as plsc

N = 10000
NP = 10240
E = 320000
D = 128
NC = 2
NS = 16
HALF = NP // 2
ACC_ROWS = HALF + 128
STRIPE = HALF // NS
EDGES_PER_TILE = E // NS
CHUNK = 128
EDGES_PAD = 20480
NCHUNK = EDGES_PAD // CHUNK
NBUF = 4
NITER = NCHUNK // NBUF
PAD_NODE = 10200
DEG_ROWS = 128

BN = 1000


def _sc_segment_sums(x, eidx_flat, zeros_rows, iota128):
    mesh = plsc.VectorSubcoreMesh(
        core_axis_name="c", subcore_axis_name="s", num_cores=NC, num_subcores=NS
    )

    @functools.partial(
        pl.kernel,
        out_type=(
            jax.ShapeDtypeStruct((NC, NP, D), jnp.float32),
            jax.ShapeDtypeStruct((NC, DEG_ROWS, 128), jnp.float32),
        ),
        mesh=mesh,
        scratch_types=[
            [pltpu.VMEM((CHUNK,), jnp.int32)] * NBUF,
            [pltpu.VMEM((CHUNK,), jnp.int32)] * NBUF,
            [pltpu.VMEM((CHUNK,), jnp.int32)] * NBUF,
            [pltpu.VMEM((CHUNK, D), jnp.float32)] * NBUF,
            pltpu.VMEM((DEG_ROWS, 128), jnp.float32),
            pltpu.VMEM((128,), jnp.int32),
            pltpu.VMEM_SHARED((ACC_ROWS, D), jnp.float32),
            pltpu.VMEM_SHARED((DEG_ROWS, 128), jnp.float32),
            [pltpu.SemaphoreType.DMA] * NBUF,
            [pltpu.SemaphoreType.DMA] * NBUF,
            [pltpu.SemaphoreType.DMA] * NBUF,
        ],
        compiler_params=pltpu.CompilerParams(needs_layout_passes=False),
    )
    def seg_kernel(x_hbm, eidx_hbm, zeros_hbm, iota_hbm,
                   s_out_hbm, deg_out_hbm,
                   gidxb, sidxb, sidxm, rows, deg_v, iota_v, acc, acc_deg,
                   isem, gsem, ssem):
        cid = lax.axis_index("c")
        sid = lax.axis_index("s")
        a0 = sid * STRIPE
        d0 = sid * (DEG_ROWS // NS)
        gbase = (cid * NS + sid) * EDGES_PAD
        sbase = ((1 - cid) * NS + sid) * EDGES_PAD
        pltpu.sync_copy(zeros_hbm.at[pl.ds(0, DEG_ROWS)], deg_v)
        pltpu.sync_copy(zeros_hbm.at[pl.ds(0, DEG_ROWS // NS)],
                        acc_deg.at[pl.ds(d0, DEG_ROWS // NS)])
        pltpu.sync_copy(iota_hbm, iota_v)

        ones16 = jnp.ones((16,), jnp.float32)

        for p in range(2):
            pltpu.sync_copy(zeros_hbm.at[pl.ds(0, STRIPE)],
                            acc.at[pl.ds(a0, STRIPE)])
            plsc.subcore_barrier()

            for b in range(NBUF):
                pltpu.sync_copy(eidx_hbm.at[pl.ds(gbase + b * CHUNK, CHUNK)],
                                gidxb[b])
                pltpu.sync_copy(eidx_hbm.at[pl.ds(sbase + b * CHUNK, CHUNK)],
                                sidxb[b])
                pltpu.async_copy(x_hbm.at[gidxb[b]], rows[b], gsem[b])

            def body(i, carry):
                for b in range(NBUF):
                    j = i * NBUF + b
                    for l in range(CHUNK // 16):
                        idx = sidxb[b][pl.ds(l * 16, 16)]
                        trash = HALF + lax.bitwise_and(idx, 127)
                        off = idx - p * HALF
                        inr = (off >= 0) & (off < HALF)
                        sidxm[b][pl.ds(l * 16, 16)] = jnp.where(inr, off, trash)
                    pltpu.make_async_copy(x_hbm.at[gidxb[b]], rows[b],
                                          gsem[b]).wait()
                    cp = pltpu.async_copy(rows[b], acc.at[sidxm[b]], ssem[b],
                                          add=True)
                    if p == 0:
                        for l in range(CHUNK // 16):
                            idx = sidxb[b][pl.ds(l * 16, 16)]
                            plsc.addupdate_scatter(
                                deg_v,
                                [lax.shift_right_logical(idx, 7),
                                 lax.bitwise_and(idx, 127)],
                                ones16)
                    cp.wait()

                    @pl.when(i < NITER - 1)
                    def _():
                        nxt = (j + NBUF) * CHUNK
                        pltpu.async_copy(
                            eidx_hbm.at[pl.ds(gbase + nxt, CHUNK)],
                            gidxb[b], isem[b])
                        gwait = pltpu.async_copy(
                            eidx_hbm.at[pl.ds(sbase + nxt, CHUNK)],
                            sidxb[b], isem[b])
                        pltpu.make_async_copy(
                            eidx_hbm.at[pl.ds(gbase + nxt, CHUNK)],
                            gidxb[b], isem[b]).wait()
                        gwait.wait()
                        pltpu.async_copy(x_hbm.at[gidxb[b]], rows[b], gsem[b])
                return carry

            lax.fori_loop(0, NITER, body, 0)
            plsc.subcore_barrier()
            pltpu.sync_copy(
                acc.at[pl.ds(a0, STRIPE)],
                s_out_hbm.at[cid, pl.ds(p * HALF + a0, STRIPE)],
            )

        pltpu.sync_copy(deg_v, acc_deg.at[iota_v], add=True)
        plsc.subcore_barrier()
        pltpu.sync_copy(
            acc_deg.at[pl.ds(d0, DEG_ROWS // NS)],
            deg_out_hbm.at[cid, pl.ds(d0, DEG_ROWS // NS)],
        )

    return seg_kernel(x, eidx_flat, zeros_rows, iota128)


def _combine_body(x_ref, sa_ref, so_ref, da_ref, do_ref,
                  wst_ref, wbt_ref, wdt_ref, wa1_ref, wa2_ref,
                  bs_ref, bd_ref, ba1_ref, out_ref):
    xb = x_ref[...]
    xs = jnp.dot(xb, wst_ref[...], preferred_element_type=jnp.float32) + bs_ref[...]
    xbB = jnp.dot(xb, wbt_ref[...], preferred_element_type=jnp.float32) + bd_ref[...]
    wdt = wdt_ref[...]
    inc = jnp.dot(sa_ref[0], wdt, preferred_element_type=jnp.float32) + da_ref[...] * xbB
    outg = jnp.dot(so_ref[0], wdt, preferred_element_type=jnp.float32) + do_ref[...] * xbB
    wa1 = wa1_ref[...]
    wa2 = wa2_ref[...]
    ba1 = ba1_ref[...]

    def logit(r):
        h = jnp.tanh(jnp.dot(r, wa1, preferred_element_type=jnp.float32) + ba1)
        return jnp.sum(h * wa2, axis=1, keepdims=True)

    l0, l1, l2 = logit(xs), logit(inc), logit(outg)
    m = jnp.maximum(jnp.maximum(l0, l1), l2)
    e0 = jnp.exp(l0 - m)
    e1 = jnp.exp(l1 - m)
    e2 = jnp.exp(l2 - m)
    out_ref[...] = (e0 * xs + e1 * inc + e2 * outg) / (e0 + e1 + e2)


def _tc_combine(x, S, din, dout, WsT, WbT, WdT, Wa1T, wa2, bs, bd, ba1):
    grid = (N // BN,)
    wspec = lambda shape: pl.BlockSpec(shape, lambda i: tuple(0 for _ in shape))
    return pl.pallas_call(
        _combine_body,
        grid=grid,
        in_specs=[
            pl.BlockSpec((BN, D), lambda i: (i, 0)),
            pl.BlockSpec((1, BN, D), lambda i: (0, i, 0)),
            pl.BlockSpec((1, BN, D), lambda i: (1, i, 0)),
            pl.BlockSpec((BN, 1), lambda i: (i, 0)),
            pl.BlockSpec((BN, 1), lambda i: (i, 0)),
            wspec((D, D)),
            wspec((D, D)),
            wspec((D, D)),
            wspec((D, 16)),
            wspec((1, 16)),
            wspec((1, D)),
            wspec((1, D)),
            wspec((1, 16)),
        ],
        out_specs=pl.BlockSpec((BN, D), lambda i: (i, 0)),
        out_shape=jax.ShapeDtypeStruct((N, D), jnp.float32),
    )(x, S, S, din, dout, WsT, WbT, WdT, Wa1T, wa2, bs, bd, ba1)


def kernel(x, edge_index, W_self, b_self, W_disc, b_disc, W_att1, b_att1, W_att2):
    x = x.astype(jnp.float32)
    xp = jnp.concatenate([x, jnp.zeros((NP - N, D), jnp.float32)], axis=0)
    e3 = edge_index.astype(jnp.int32).reshape(2, NS, EDGES_PER_TILE)
    epad = jnp.full((2, NS, EDGES_PAD - EDGES_PER_TILE), PAD_NODE, jnp.int32)
    eidx4 = jnp.concatenate([e3, epad], axis=2).reshape(2 * NS * EDGES_PAD)
    zeros_rows = jnp.zeros((STRIPE, D), jnp.float32)
    iota128 = jnp.arange(128, dtype=jnp.int32)
    S, degH = _sc_segment_sums(xp, eidx4, zeros_rows, iota128)
    deg = degH.reshape(NC, DEG_ROWS * 128)[:, :N]
    din = deg[0].reshape(N, 1)
    dout = deg[1].reshape(N, 1)

    Wa = W_disc[:, :D]
    Wb = W_disc[:, D:]
    WsT = W_self.T
    WbT = Wb.T
    WdT = (Wa - Wb).T
    Wa1T = W_att1.T
    wa2 = W_att2.reshape(1, 16)
    bs = b_self.reshape(1, D)
    bd = b_disc.reshape(1, D)
    ba1 = b_att1.reshape(1, 16)
    return _tc_combine(x, S, din, dout, WsT, WbT, WdT, Wa1T, wa2, bs, bd, ba1)


# speedup vs baseline: 2.4489x; 2.4489x over previous
import functools

import jax
import jax.numpy as jnp
from jax import lax
from jax.experimental import pallas as pl
from jax.experimental.pallas import tpu as pltpu
from jax.experimental.pallas import tpu_sc as plsc

N = 10000
NP = 10240
E = 320000
D = 128
NC = 2
NS = 16
ROWS_PER_TILE = NP // NS
EDGES_PER_TILE = E // NS
CHUNK = 80
NCHUNK = EDGES_PER_TILE // CHUNK
PAD_NODE = 10200
DEG_ROWS = 128

BN = 1000


def _sc_segment_sums(x, eidx_flat, zeros_rows, iota128):
    mesh = plsc.VectorSubcoreMesh(
        core_axis_name="c", subcore_axis_name="s", num_cores=NC, num_subcores=NS
    )

    @functools.partial(
        pl.kernel,
        out_type=(
            jax.ShapeDtypeStruct((NC, NP, D), jnp.float32),
            jax.ShapeDtypeStruct((NC, DEG_ROWS, 128), jnp.float32),
        ),
        mesh=mesh,
        scratch_types=[
            pltpu.VMEM((CHUNK,), jnp.int32),
            pltpu.VMEM((CHUNK,), jnp.int32),
            pltpu.VMEM((CHUNK, D), jnp.float32),
            pltpu.VMEM((DEG_ROWS, 128), jnp.float32),
            pltpu.VMEM((128,), jnp.int32),
            pltpu.VMEM_SHARED((NP, D), jnp.float32),
            pltpu.VMEM_SHARED((DEG_ROWS, 128), jnp.float32),
        ],
        compiler_params=pltpu.CompilerParams(needs_layout_passes=False),
    )
    def seg_kernel(x_hbm, eidx_hbm, zeros_hbm, iota_hbm,
                   s_out_hbm, deg_out_hbm,
                   gidx, sidx, rows, deg_v, iota_v, acc, acc_deg):
        cid = lax.axis_index("c")
        sid = lax.axis_index("s")
        r0 = sid * ROWS_PER_TILE
        d0 = sid * (DEG_ROWS // NS)
        gbase = (cid * NS + sid) * EDGES_PER_TILE
        sbase = ((1 - cid) * NS + sid) * EDGES_PER_TILE
        pltpu.sync_copy(zeros_hbm.at[pl.ds(0, DEG_ROWS)], deg_v)
        pltpu.sync_copy(zeros_hbm.at[pl.ds(0, DEG_ROWS // NS)],
                        acc_deg.at[pl.ds(d0, DEG_ROWS // NS)])
        pltpu.sync_copy(iota_hbm, iota_v)
        pltpu.sync_copy(zeros_hbm, acc.at[pl.ds(r0, ROWS_PER_TILE)])
        plsc.subcore_barrier()

        ones16 = jnp.ones((16,), jnp.float32)

        def body(i, carry):
            pltpu.sync_copy(eidx_hbm.at[pl.ds(gbase + i * CHUNK, CHUNK)], gidx)
            pltpu.sync_copy(eidx_hbm.at[pl.ds(sbase + i * CHUNK, CHUNK)], sidx)
            pltpu.sync_copy(x_hbm.at[gidx], rows)
            pltpu.sync_copy(rows, acc.at[sidx], add=True)
            for l in range(CHUNK // 16):
                idx = sidx[pl.ds(l * 16, 16)]
                plsc.addupdate_scatter(
                    deg_v,
                    [lax.shift_right_logical(idx, 7),
                     lax.bitwise_and(idx, 127)],
                    ones16)
            return carry

        lax.fori_loop(0, NCHUNK, body, 0)

        pltpu.sync_copy(deg_v, acc_deg.at[iota_v], add=True)
        plsc.subcore_barrier()
        pltpu.sync_copy(acc.at[pl.ds(r0, ROWS_PER_TILE)],
                        s_out_hbm.at[cid, pl.ds(r0, ROWS_PER_TILE)])
        pltpu.sync_copy(
            acc_deg.at[pl.ds(d0, DEG_ROWS // NS)],
            deg_out_hbm.at[cid, pl.ds(d0, DEG_ROWS // NS)],
        )

    return seg_kernel(x, eidx_flat, zeros_rows, iota128)


def _combine_body(x_ref, sa_ref, so_ref, da_ref, do_ref,
                  wst_ref, wbt_ref, wdt_ref, wa1_ref, wa2_ref,
                  bs_ref, bd_ref, ba1_ref, out_ref):
    xb = x_ref[...]
    xs = jnp.dot(xb, wst_ref[...], preferred_element_type=jnp.float32) + bs_ref[...]
    xbB = jnp.dot(xb, wbt_ref[...], preferred_element_type=jnp.float32) + bd_ref[...]
    wdt = wdt_ref[...]
    inc = jnp.dot(sa_ref[0], wdt, preferred_element_type=jnp.float32) + da_ref[...] * xbB
    outg = jnp.dot(so_ref[0], wdt, preferred_element_type=jnp.float32) + do_ref[...] * xbB
    wa1 = wa1_ref[...]
    wa2 = wa2_ref[...]
    ba1 = ba1_ref[...]

    def logit(r):
        h = jnp.tanh(jnp.dot(r, wa1, preferred_element_type=jnp.float32) + ba1)
        return jnp.sum(h * wa2, axis=1, keepdims=True)

    l0, l1, l2 = logit(xs), logit(inc), logit(outg)
    m = jnp.maximum(jnp.maximum(l0, l1), l2)
    e0 = jnp.exp(l0 - m)
    e1 = jnp.exp(l1 - m)
    e2 = jnp.exp(l2 - m)
    out_ref[...] = (e0 * xs + e1 * inc + e2 * outg) / (e0 + e1 + e2)


def _tc_combine(x, S, din, dout, WsT, WbT, WdT, Wa1T, wa2, bs, bd, ba1):
    grid = (N // BN,)
    wspec = lambda shape: pl.BlockSpec(shape, lambda i: tuple(0 for _ in shape))
    return pl.pallas_call(
        _combine_body,
        grid=grid,
        in_specs=[
            pl.BlockSpec((BN, D), lambda i: (i, 0)),
            pl.BlockSpec((1, BN, D), lambda i: (0, i, 0)),
            pl.BlockSpec((1, BN, D), lambda i: (1, i, 0)),
            pl.BlockSpec((BN, 1), lambda i: (i, 0)),
            pl.BlockSpec((BN, 1), lambda i: (i, 0)),
            wspec((D, D)),
            wspec((D, D)),
            wspec((D, D)),
            wspec((D, 16)),
            wspec((1, 16)),
            wspec((1, D)),
            wspec((1, D)),
            wspec((1, 16)),
        ],
        out_specs=pl.BlockSpec((BN, D), lambda i: (i, 0)),
        out_shape=jax.ShapeDtypeStruct((N, D), jnp.float32),
    )(x, S, S, din, dout, WsT, WbT, WdT, Wa1T, wa2, bs, bd, ba1)


def kernel(x, edge_index, W_self, b_self, W_disc, b_disc, W_att1, b_att1, W_att2):
    x = x.astype(jnp.float32)
    xp = jnp.concatenate([x, jnp.zeros((NP - N, D), jnp.float32)], axis=0)
    eidx = edge_index.astype(jnp.int32).reshape(2 * NS * EDGES_PER_TILE)
    zeros_rows = jnp.zeros((ROWS_PER_TILE, D), jnp.float32)
    iota128 = jnp.arange(128, dtype=jnp.int32)
    S, degH = _sc_segment_sums(xp, eidx, zeros_rows, iota128)
    deg = degH.reshape(NC, DEG_ROWS * 128)[:, :N]
    din = deg[0].reshape(N, 1)
    dout = deg[1].reshape(N, 1)

    Wa = W_disc[:, :D]
    Wb = W_disc[:, D:]
    WsT = W_self.T
    WbT = Wb.T
    WdT = (Wa - Wb).T
    Wa1T = W_att1.T
    wa2 = W_att2.reshape(1, 16)
    bs = b_self.reshape(1, D)
    bd = b_disc.reshape(1, D)
    ba1 = b_att1.reshape(1, 16)
    return _tc_combine(x, S, din, dout, WsT, WbT, WdT, Wa1T, wa2, bs, bd, ba1)


# speedup vs baseline: 3.9551x; 1.6150x over previous
import functools

import jax
import jax.numpy as jnp
from jax import lax
from jax.experimental import pallas as pl
from jax.experimental.pallas import tpu as pltpu
from jax.experimental.pallas import tpu_sc as plsc

N = 10000
NP = 10240
E = 320000
D = 128
NC = 2
NS = 16
ROWS_PER_TILE = NP // NS
EDGES_PER_TILE = E // NS
CHUNK = 80
NCHUNK = EDGES_PER_TILE // CHUNK
PAD_NODE = 10200
DEG_ROWS = 128

BN = 1000


def _sc_segment_sums(x, eidx_flat, zeros_rows, iota128):
    mesh = plsc.VectorSubcoreMesh(
        core_axis_name="c", subcore_axis_name="s", num_cores=NC, num_subcores=NS
    )

    @functools.partial(
        pl.kernel,
        out_type=(
            jax.ShapeDtypeStruct((NC, NP, D), jnp.float32),
            jax.ShapeDtypeStruct((NC, DEG_ROWS, 128), jnp.float32),
        ),
        mesh=mesh,
        scratch_types=[
            [pltpu.VMEM((CHUNK,), jnp.int32)] * 2,
            [pltpu.VMEM((CHUNK,), jnp.int32)] * 2,
            [pltpu.VMEM((CHUNK, D), jnp.float32)] * 2,
            pltpu.VMEM((DEG_ROWS, 128), jnp.float32),
            pltpu.VMEM((128,), jnp.int32),
            pltpu.VMEM_SHARED((NP, D), jnp.float32),
            pltpu.VMEM_SHARED((DEG_ROWS, 128), jnp.float32),
            [pltpu.SemaphoreType.DMA] * 2,
        ],
        compiler_params=pltpu.CompilerParams(needs_layout_passes=False),
    )
    def seg_kernel(x_hbm, eidx_hbm, zeros_hbm, iota_hbm,
                   s_out_hbm, deg_out_hbm,
                   gidx, sidx, rows, deg_v, iota_v, acc, acc_deg, gsem):
        cid = lax.axis_index("c")
        sid = lax.axis_index("s")
        r0 = sid * ROWS_PER_TILE
        d0 = sid * (DEG_ROWS // NS)
        gbase = (cid * NS + sid) * EDGES_PER_TILE
        sbase = ((1 - cid) * NS + sid) * EDGES_PER_TILE
        pltpu.sync_copy(zeros_hbm.at[pl.ds(0, DEG_ROWS)], deg_v)
        pltpu.sync_copy(zeros_hbm.at[pl.ds(0, DEG_ROWS // NS)],
                        acc_deg.at[pl.ds(d0, DEG_ROWS // NS)])
        pltpu.sync_copy(iota_hbm, iota_v)
        pltpu.sync_copy(zeros_hbm, acc.at[pl.ds(r0, ROWS_PER_TILE)])
        plsc.subcore_barrier()

        ones16 = jnp.ones((16,), jnp.float32)

        pltpu.sync_copy(eidx_hbm.at[pl.ds(gbase, CHUNK)], gidx[0])
        pltpu.sync_copy(eidx_hbm.at[pl.ds(sbase, CHUNK)], sidx[0])
        pltpu.async_copy(x_hbm.at[gidx[0]], rows[0], gsem[0])

        def body(i, carry):
            for b in range(2):
                j = 2 * i + b
                nb = 1 - b

                @pl.when(j < NCHUNK - 1)
                def _():
                    pltpu.sync_copy(
                        eidx_hbm.at[pl.ds(gbase + (j + 1) * CHUNK, CHUNK)],
                        gidx[nb])
                    pltpu.sync_copy(
                        eidx_hbm.at[pl.ds(sbase + (j + 1) * CHUNK, CHUNK)],
                        sidx[nb])
                    pltpu.async_copy(x_hbm.at[gidx[nb]], rows[nb], gsem[nb])

                pltpu.make_async_copy(x_hbm.at[gidx[b]], rows[b],
                                      gsem[b]).wait()
                pltpu.sync_copy(rows[b], acc.at[sidx[b]], add=True)
                for l in range(CHUNK // 16):
                    idx = sidx[b][pl.ds(l * 16, 16)]
                    plsc.addupdate_scatter(
                        deg_v,
                        [lax.shift_right_logical(idx, 7),
                         lax.bitwise_and(idx, 127)],
                        ones16)
            return carry

        lax.fori_loop(0, NCHUNK // 2, body, 0)

        pltpu.sync_copy(deg_v, acc_deg.at[iota_v], add=True)
        plsc.subcore_barrier()
        pltpu.sync_copy(acc.at[pl.ds(r0, ROWS_PER_TILE)],
                        s_out_hbm.at[cid, pl.ds(r0, ROWS_PER_TILE)])
        pltpu.sync_copy(
            acc_deg.at[pl.ds(d0, DEG_ROWS // NS)],
            deg_out_hbm.at[cid, pl.ds(d0, DEG_ROWS // NS)],
        )

    return seg_kernel(x, eidx_flat, zeros_rows, iota128)


def _combine_body(x_ref, sa_ref, so_ref, da_ref, do_ref,
                  wst_ref, wbt_ref, wdt_ref, wa1_ref, wa2_ref,
                  bs_ref, bd_ref, ba1_ref, out_ref):
    xb = x_ref[...]
    xs = jnp.dot(xb, wst_ref[...], preferred_element_type=jnp.float32) + bs_ref[...]
    xbB = jnp.dot(xb, wbt_ref[...], preferred_element_type=jnp.float32) + bd_ref[...]
    wdt = wdt_ref[...]
    inc = jnp.dot(sa_ref[0], wdt, preferred_element_type=jnp.float32) + da_ref[...] * xbB
    outg = jnp.dot(so_ref[0], wdt, preferred_element_type=jnp.float32) + do_ref[...] * xbB
    wa1 = wa1_ref[...]
    wa2 = wa2_ref[...]
    ba1 = ba1_ref[...]

    def logit(r):
        h = jnp.tanh(jnp.dot(r, wa1, preferred_element_type=jnp.float32) + ba1)
        return jnp.sum(h * wa2, axis=1, keepdims=True)

    l0, l1, l2 = logit(xs), logit(inc), logit(outg)
    m = jnp.maximum(jnp.maximum(l0, l1), l2)
    e0 = jnp.exp(l0 - m)
    e1 = jnp.exp(l1 - m)
    e2 = jnp.exp(l2 - m)
    out_ref[...] = (e0 * xs + e1 * inc + e2 * outg) / (e0 + e1 + e2)


def _tc_combine(x, S, din, dout, WsT, WbT, WdT, Wa1T, wa2, bs, bd, ba1):
    grid = (N // BN,)
    wspec = lambda shape: pl.BlockSpec(shape, lambda i: tuple(0 for _ in shape))
    return pl.pallas_call(
        _combine_body,
        grid=grid,
        in_specs=[
            pl.BlockSpec((BN, D), lambda i: (i, 0)),
            pl.BlockSpec((1, BN, D), lambda i: (0, i, 0)),
            pl.BlockSpec((1, BN, D), lambda i: (1, i, 0)),
            pl.BlockSpec((BN, 1), lambda i: (i, 0)),
            pl.BlockSpec((BN, 1), lambda i: (i, 0)),
            wspec((D, D)),
            wspec((D, D)),
            wspec((D, D)),
            wspec((D, 16)),
            wspec((1, 16)),
            wspec((1, D)),
            wspec((1, D)),
            wspec((1, 16)),
        ],
        out_specs=pl.BlockSpec((BN, D), lambda i: (i, 0)),
        out_shape=jax.ShapeDtypeStruct((N, D), jnp.float32),
    )(x, S, S, din, dout, WsT, WbT, WdT, Wa1T, wa2, bs, bd, ba1)


def kernel(x, edge_index, W_self, b_self, W_disc, b_disc, W_att1, b_att1, W_att2):
    x = x.astype(jnp.float32)
    xp = jnp.concatenate([x, jnp.zeros((NP - N, D), jnp.float32)], axis=0)
    eidx = edge_index.astype(jnp.int32).reshape(2 * NS * EDGES_PER_TILE)
    zeros_rows = jnp.zeros((ROWS_PER_TILE, D), jnp.float32)
    iota128 = jnp.arange(128, dtype=jnp.int32)
    S, degH = _sc_segment_sums(xp, eidx, zeros_rows, iota128)
    deg = degH.reshape(NC, DEG_ROWS * 128)[:, :N]
    din = deg[0].reshape(N, 1)
    dout = deg[1].reshape(N, 1)

    Wa = W_disc[:, :D]
    Wb = W_disc[:, D:]
    WsT = W_self.T
    WbT = Wb.T
    WdT = (Wa - Wb).T
    Wa1T = W_att1.T
    wa2 = W_att2.reshape(1, 16)
    bs = b_self.reshape(1, D)
    bd = b_disc.reshape(1, D)
    ba1 = b_att1.reshape(1, 16)
    return _tc_combine(x, S, din, dout, WsT, WbT, WdT, Wa1T, wa2, bs, bd, ba1)


# speedup vs baseline: 4.0072x; 1.0132x over previous
import functools

import jax
import jax.numpy as jnp
from jax import lax
from jax.experimental import pallas as pl
from jax.experimental.pallas import tpu as pltpu
from jax.experimental.pallas import tpu_sc as plsc

N = 10000
NP = 10240
E = 320000
D = 128
NC = 2
NS = 16
ROWS_PER_TILE = NP // NS
EDGES_PER_TILE = E // NS
CHUNK = 80
NCHUNK = EDGES_PER_TILE // CHUNK
PAD_NODE = 10200
DEG_ROWS = 128

BN = 1000


def _sc_segment_sums(x, eidx_flat, zeros_rows, iota128):
    mesh = plsc.VectorSubcoreMesh(
        core_axis_name="c", subcore_axis_name="s", num_cores=NC, num_subcores=NS
    )

    @functools.partial(
        pl.kernel,
        out_type=(
            jax.ShapeDtypeStruct((NC, NP, D), jnp.float32),
            jax.ShapeDtypeStruct((NC, DEG_ROWS, 128), jnp.float32),
        ),
        mesh=mesh,
        scratch_types=[
            [pltpu.VMEM((CHUNK,), jnp.int32)] * 2,
            [pltpu.VMEM((CHUNK,), jnp.int32)] * 2,
            [pltpu.VMEM((CHUNK, D), jnp.float32)] * 2,
            pltpu.VMEM((DEG_ROWS, 128), jnp.float32),
            pltpu.VMEM((128,), jnp.int32),
            pltpu.VMEM_SHARED((NP, D), jnp.float32),
            pltpu.VMEM_SHARED((DEG_ROWS, 128), jnp.float32),
            [pltpu.SemaphoreType.DMA] * 2,
            [pltpu.SemaphoreType.DMA] * 2,
        ],
        compiler_params=pltpu.CompilerParams(needs_layout_passes=False),
    )
    def seg_kernel(x_hbm, eidx_hbm, zeros_hbm, iota_hbm,
                   s_out_hbm, deg_out_hbm,
                   gidx, sidx, rows, deg_v, iota_v, acc, acc_deg, gsem, ssem):
        cid = lax.axis_index("c")
        sid = lax.axis_index("s")
        r0 = sid * ROWS_PER_TILE
        d0 = sid * (DEG_ROWS // NS)
        gbase = (cid * NS + sid) * EDGES_PER_TILE
        sbase = ((1 - cid) * NS + sid) * EDGES_PER_TILE
        pltpu.sync_copy(zeros_hbm.at[pl.ds(0, DEG_ROWS)], deg_v)
        pltpu.sync_copy(zeros_hbm.at[pl.ds(0, DEG_ROWS // NS)],
                        acc_deg.at[pl.ds(d0, DEG_ROWS // NS)])
        pltpu.sync_copy(iota_hbm, iota_v)
        pltpu.sync_copy(zeros_hbm, acc.at[pl.ds(r0, ROWS_PER_TILE)])
        plsc.subcore_barrier()

        ones16 = jnp.ones((16,), jnp.float32)

        pltpu.sync_copy(eidx_hbm.at[pl.ds(gbase, CHUNK)], gidx[0])
        pltpu.sync_copy(eidx_hbm.at[pl.ds(sbase, CHUNK)], sidx[0])
        pltpu.async_copy(x_hbm.at[gidx[0]], rows[0], gsem[0])

        def body(i, carry):
            for b in range(2):
                j = 2 * i + b
                nb = 1 - b

                @pl.when(j < NCHUNK - 1)
                def _():
                    @pl.when(j > 0)
                    def _():
                        pltpu.make_async_copy(rows[nb], acc.at[sidx[nb]],
                                              ssem[nb]).wait()
                    pltpu.sync_copy(
                        eidx_hbm.at[pl.ds(gbase + (j + 1) * CHUNK, CHUNK)],
                        gidx[nb])
                    pltpu.sync_copy(
                        eidx_hbm.at[pl.ds(sbase + (j + 1) * CHUNK, CHUNK)],
                        sidx[nb])
                    pltpu.async_copy(x_hbm.at[gidx[nb]], rows[nb], gsem[nb])

                pltpu.make_async_copy(x_hbm.at[gidx[b]], rows[b],
                                      gsem[b]).wait()
                pltpu.async_copy(rows[b], acc.at[sidx[b]], ssem[b], add=True)
                for l in range(CHUNK // 16):
                    idx = sidx[b][pl.ds(l * 16, 16)]
                    plsc.addupdate_scatter(
                        deg_v,
                        [lax.shift_right_logical(idx, 7),
                         lax.bitwise_and(idx, 127)],
                        ones16)
            return carry

        lax.fori_loop(0, NCHUNK // 2, body, 0)
        pltpu.make_async_copy(rows[0], acc.at[sidx[0]], ssem[0]).wait()
        pltpu.make_async_copy(rows[1], acc.at[sidx[1]], ssem[1]).wait()

        pltpu.sync_copy(deg_v, acc_deg.at[iota_v], add=True)
        plsc.subcore_barrier()
        pltpu.sync_copy(acc.at[pl.ds(r0, ROWS_PER_TILE)],
                        s_out_hbm.at[cid, pl.ds(r0, ROWS_PER_TILE)])
        pltpu.sync_copy(
            acc_deg.at[pl.ds(d0, DEG_ROWS // NS)],
            deg_out_hbm.at[cid, pl.ds(d0, DEG_ROWS // NS)],
        )

    return seg_kernel(x, eidx_flat, zeros_rows, iota128)


def _combine_body(x_ref, sa_ref, so_ref, da_ref, do_ref,
                  wst_ref, wbt_ref, wdt_ref, wa1_ref, wa2_ref,
                  bs_ref, bd_ref, ba1_ref, out_ref):
    xb = x_ref[...]
    xs = jnp.dot(xb, wst_ref[...], preferred_element_type=jnp.float32) + bs_ref[...]
    xbB = jnp.dot(xb, wbt_ref[...], preferred_element_type=jnp.float32) + bd_ref[...]
    wdt = wdt_ref[...]
    inc = jnp.dot(sa_ref[0], wdt, preferred_element_type=jnp.float32) + da_ref[...] * xbB
    outg = jnp.dot(so_ref[0], wdt, preferred_element_type=jnp.float32) + do_ref[...] * xbB
    wa1 = wa1_ref[...]
    wa2 = wa2_ref[...]
    ba1 = ba1_ref[...]

    def logit(r):
        h = jnp.tanh(jnp.dot(r, wa1, preferred_element_type=jnp.float32) + ba1)
        return jnp.sum(h * wa2, axis=1, keepdims=True)

    l0, l1, l2 = logit(xs), logit(inc), logit(outg)
    m = jnp.maximum(jnp.maximum(l0, l1), l2)
    e0 = jnp.exp(l0 - m)
    e1 = jnp.exp(l1 - m)
    e2 = jnp.exp(l2 - m)
    out_ref[...] = (e0 * xs + e1 * inc + e2 * outg) / (e0 + e1 + e2)


def _tc_combine(x, S, din, dout, WsT, WbT, WdT, Wa1T, wa2, bs, bd, ba1):
    grid = (N // BN,)
    wspec = lambda shape: pl.BlockSpec(shape, lambda i: tuple(0 for _ in shape))
    return pl.pallas_call(
        _combine_body,
        grid=grid,
        in_specs=[
            pl.BlockSpec((BN, D), lambda i: (i, 0)),
            pl.BlockSpec((1, BN, D), lambda i: (0, i, 0)),
            pl.BlockSpec((1, BN, D), lambda i: (1, i, 0)),
            pl.BlockSpec((BN, 1), lambda i: (i, 0)),
            pl.BlockSpec((BN, 1), lambda i: (i, 0)),
            wspec((D, D)),
            wspec((D, D)),
            wspec((D, D)),
            wspec((D, 16)),
            wspec((1, 16)),
            wspec((1, D)),
            wspec((1, D)),
            wspec((1, 16)),
        ],
        out_specs=pl.BlockSpec((BN, D), lambda i: (i, 0)),
        out_shape=jax.ShapeDtypeStruct((N, D), jnp.float32),
    )(x, S, S, din, dout, WsT, WbT, WdT, Wa1T, wa2, bs, bd, ba1)


def kernel(x, edge_index, W_self, b_self, W_disc, b_disc, W_att1, b_att1, W_att2):
    x = x.astype(jnp.float32)
    xp = jnp.concatenate([x, jnp.zeros((NP - N, D), jnp.float32)], axis=0)
    eidx = edge_index.astype(jnp.int32).reshape(2 * NS * EDGES_PER_TILE)
    zeros_rows = jnp.zeros((ROWS_PER_TILE, D), jnp.float32)
    iota128 = jnp.arange(128, dtype=jnp.int32)
    S, degH = _sc_segment_sums(xp, eidx, zeros_rows, iota128)
    deg = degH.reshape(NC, DEG_ROWS * 128)[:, :N]
    din = deg[0].reshape(N, 1)
    dout = deg[1].reshape(N, 1)

    Wa = W_disc[:, :D]
    Wb = W_disc[:, D:]
    WsT = W_self.T
    WbT = Wb.T
    WdT = (Wa - Wb).T
    Wa1T = W_att1.T
    wa2 = W_att2.reshape(1, 16)
    bs = b_self.reshape(1, D)
    bd = b_disc.reshape(1, D)
    ba1 = b_att1.reshape(1, 16)
    return _tc_combine(x, S, din, dout, WsT, WbT, WdT, Wa1T, wa2, bs, bd, ba1)


# speedup vs baseline: 5.0213x; 1.2531x over previous
import functools

import jax
import jax.numpy as jnp
from jax import lax
from jax.experimental import pallas as pl
from jax.experimental.pallas import tpu as pltpu
from jax.experimental.pallas import tpu_sc as plsc

N = 10000
NP = 10240
E = 320000
D = 128
NC = 2
NS = 16
ROWS_PER_TILE = NP // NS
EDGES_PER_TILE = E // NS
CHUNK = 80
NCHUNK = EDGES_PER_TILE // CHUNK
CPB = 10
IBLK = CHUNK * CPB
NBLK = EDGES_PER_TILE // IBLK
PAD_NODE = 10200
DEG_ROWS = 128

BN = 1000


def _sc_segment_sums(x, eidx_flat, zeros_rows, iota128):
    mesh = plsc.VectorSubcoreMesh(
        core_axis_name="c", subcore_axis_name="s", num_cores=NC, num_subcores=NS
    )

    @functools.partial(
        pl.kernel,
        out_type=(
            jax.ShapeDtypeStruct((NC, NP, D), jnp.float32),
            jax.ShapeDtypeStruct((NC, DEG_ROWS, 128), jnp.float32),
        ),
        mesh=mesh,
        scratch_types=[
            pltpu.VMEM((IBLK,), jnp.int32),
            pltpu.VMEM((IBLK,), jnp.int32),
            [pltpu.VMEM((CHUNK, D), jnp.float32)] * 2,
            pltpu.VMEM((DEG_ROWS, 128), jnp.float32),
            pltpu.VMEM((128,), jnp.int32),
            pltpu.VMEM_SHARED((NP, D), jnp.float32),
            pltpu.VMEM_SHARED((DEG_ROWS, 128), jnp.float32),
            [pltpu.SemaphoreType.DMA] * 2,
            [pltpu.SemaphoreType.DMA] * 2,
        ],
        compiler_params=pltpu.CompilerParams(needs_layout_passes=False),
    )
    def seg_kernel(x_hbm, eidx_hbm, zeros_hbm, iota_hbm,
                   s_out_hbm, deg_out_hbm,
                   gidx, sidx, rows, deg_v, iota_v, acc, acc_deg, gsem, ssem):
        cid = lax.axis_index("c")
        sid = lax.axis_index("s")
        r0 = sid * ROWS_PER_TILE
        d0 = sid * (DEG_ROWS // NS)
        gbase = (cid * NS + sid) * EDGES_PER_TILE
        sbase = ((1 - cid) * NS + sid) * EDGES_PER_TILE
        pltpu.sync_copy(zeros_hbm.at[pl.ds(0, DEG_ROWS)], deg_v)
        pltpu.sync_copy(zeros_hbm.at[pl.ds(0, DEG_ROWS // NS)],
                        acc_deg.at[pl.ds(d0, DEG_ROWS // NS)])
        pltpu.sync_copy(iota_hbm, iota_v)
        pltpu.sync_copy(zeros_hbm, acc.at[pl.ds(r0, ROWS_PER_TILE)])
        plsc.subcore_barrier()

        ones16 = jnp.ones((16,), jnp.float32)

        def blk(kb, carry):
            pltpu.sync_copy(eidx_hbm.at[pl.ds(gbase + kb * IBLK, IBLK)],
                            gidx)
            pltpu.sync_copy(eidx_hbm.at[pl.ds(sbase + kb * IBLK, IBLK)],
                            sidx)
            gs = [gidx.at[pl.ds(j * CHUNK, CHUNK)] for j in range(CPB)]
            ss = [sidx.at[pl.ds(j * CHUNK, CHUNK)] for j in range(CPB)]
            pltpu.async_copy(x_hbm.at[gs[0]], rows[0], gsem[0])
            for j in range(CPB):
                b = j & 1
                nb = 1 - b
                if j < CPB - 1:
                    if j > 0:
                        pltpu.make_async_copy(rows[nb], acc.at[ss[j - 1]],
                                              ssem[nb]).wait()
                    pltpu.async_copy(x_hbm.at[gs[j + 1]], rows[nb], gsem[nb])
                pltpu.make_async_copy(x_hbm.at[gs[j]], rows[b],
                                      gsem[b]).wait()
                pltpu.async_copy(rows[b], acc.at[ss[j]], ssem[b], add=True)
                for l in range(CHUNK // 16):
                    idx = sidx[pl.ds(j * CHUNK + l * 16, 16)]
                    plsc.addupdate_scatter(
                        deg_v,
                        [lax.shift_right_logical(idx, 7),
                         lax.bitwise_and(idx, 127)],
                        ones16)
            pltpu.make_async_copy(rows[0], acc.at[ss[CPB - 2]],
                                  ssem[0]).wait()
            pltpu.make_async_copy(rows[1], acc.at[ss[CPB - 1]],
                                  ssem[1]).wait()
            return carry

        lax.fori_loop(0, NBLK, blk, 0)

        pltpu.sync_copy(deg_v, acc_deg.at[iota_v], add=True)
        plsc.subcore_barrier()
        pltpu.sync_copy(acc.at[pl.ds(r0, ROWS_PER_TILE)],
                        s_out_hbm.at[cid, pl.ds(r0, ROWS_PER_TILE)])
        pltpu.sync_copy(
            acc_deg.at[pl.ds(d0, DEG_ROWS // NS)],
            deg_out_hbm.at[cid, pl.ds(d0, DEG_ROWS // NS)],
        )

    return seg_kernel(x, eidx_flat, zeros_rows, iota128)


def _combine_body(x_ref, sa_ref, so_ref, da_ref, do_ref,
                  wst_ref, wbt_ref, wdt_ref, wa1_ref, wa2_ref,
                  bs_ref, bd_ref, ba1_ref, out_ref):
    xb = x_ref[...]
    xs = jnp.dot(xb, wst_ref[...], preferred_element_type=jnp.float32) + bs_ref[...]
    xbB = jnp.dot(xb, wbt_ref[...], preferred_element_type=jnp.float32) + bd_ref[...]
    wdt = wdt_ref[...]
    inc = jnp.dot(sa_ref[0], wdt, preferred_element_type=jnp.float32) + da_ref[...] * xbB
    outg = jnp.dot(so_ref[0], wdt, preferred_element_type=jnp.float32) + do_ref[...] * xbB
    wa1 = wa1_ref[...]
    wa2 = wa2_ref[...]
    ba1 = ba1_ref[...]

    def logit(r):
        h = jnp.tanh(jnp.dot(r, wa1, preferred_element_type=jnp.float32) + ba1)
        return jnp.sum(h * wa2, axis=1, keepdims=True)

    l0, l1, l2 = logit(xs), logit(inc), logit(outg)
    m = jnp.maximum(jnp.maximum(l0, l1), l2)
    e0 = jnp.exp(l0 - m)
    e1 = jnp.exp(l1 - m)
    e2 = jnp.exp(l2 - m)
    out_ref[...] = (e0 * xs + e1 * inc + e2 * outg) / (e0 + e1 + e2)


def _tc_combine(x, S, din, dout, WsT, WbT, WdT, Wa1T, wa2, bs, bd, ba1):
    grid = (N // BN,)
    wspec = lambda shape: pl.BlockSpec(shape, lambda i: tuple(0 for _ in shape))
    return pl.pallas_call(
        _combine_body,
        grid=grid,
        in_specs=[
            pl.BlockSpec((BN, D), lambda i: (i, 0)),
            pl.BlockSpec((1, BN, D), lambda i: (0, i, 0)),
            pl.BlockSpec((1, BN, D), lambda i: (1, i, 0)),
            pl.BlockSpec((BN, 1), lambda i: (i, 0)),
            pl.BlockSpec((BN, 1), lambda i: (i, 0)),
            wspec((D, D)),
            wspec((D, D)),
            wspec((D, D)),
            wspec((D, 16)),
            wspec((1, 16)),
            wspec((1, D)),
            wspec((1, D)),
            wspec((1, 16)),
        ],
        out_specs=pl.BlockSpec((BN, D), lambda i: (i, 0)),
        out_shape=jax.ShapeDtypeStruct((N, D), jnp.float32),
    )(x, S, S, din, dout, WsT, WbT, WdT, Wa1T, wa2, bs, bd, ba1)


def kernel(x, edge_index, W_self, b_self, W_disc, b_disc, W_att1, b_att1, W_att2):
    x = x.astype(jnp.float32)
    xp = jnp.concatenate([x, jnp.zeros((NP - N, D), jnp.float32)], axis=0)
    eidx = edge_index.astype(jnp.int32).reshape(2 * NS * EDGES_PER_TILE)
    zeros_rows = jnp.zeros((ROWS_PER_TILE, D), jnp.float32)
    iota128 = jnp.arange(128, dtype=jnp.int32)
    S, degH = _sc_segment_sums(xp, eidx, zeros_rows, iota128)
    deg = degH.reshape(NC, DEG_ROWS * 128)[:, :N]
    din = deg[0].reshape(N, 1)
    dout = deg[1].reshape(N, 1)

    Wa = W_disc[:, :D]
    Wb = W_disc[:, D:]
    WsT = W_self.T
    WbT = Wb.T
    WdT = (Wa - Wb).T
    Wa1T = W_att1.T
    wa2 = W_att2.reshape(1, 16)
    bs = b_self.reshape(1, D)
    bd = b_disc.reshape(1, D)
    ba1 = b_att1.reshape(1, 16)
    return _tc_combine(x, S, din, dout, WsT, WbT, WdT, Wa1T, wa2, bs, bd, ba1)


# speedup vs baseline: 5.4917x; 1.0937x over previous
import functools

import jax
import jax.numpy as jnp
from jax import lax
from jax.experimental import pallas as pl
from jax.experimental.pallas import tpu as pltpu
from jax.experimental.pallas import tpu_sc as plsc

N = 10000
NP = 10240
E = 320000
D = 128
NC = 2
NS = 16
ROWS_PER_TILE = NP // NS
EDGES_PER_TILE = E // NS
CHUNK = 80
NCHUNK = EDGES_PER_TILE // CHUNK
CPB = 25
IBLK = CHUNK * CPB
NBLK = EDGES_PER_TILE // IBLK
JE0 = CPB - 1 if (CPB - 1) % 2 == 0 else CPB - 2
JE1 = CPB - 1 if (CPB - 1) % 2 == 1 else CPB - 2
PAD_NODE = 10200
DEG_ROWS = 128

BN = 1000


def _sc_segment_sums(x, eidx_flat, zeros_rows, iota128):
    mesh = plsc.VectorSubcoreMesh(
        core_axis_name="c", subcore_axis_name="s", num_cores=NC, num_subcores=NS
    )

    @functools.partial(
        pl.kernel,
        out_type=(
            jax.ShapeDtypeStruct((NC, NP, D), jnp.float32),
            jax.ShapeDtypeStruct((NC, DEG_ROWS, 128), jnp.float32),
        ),
        mesh=mesh,
        scratch_types=[
            pltpu.VMEM((IBLK,), jnp.int32),
            pltpu.VMEM((IBLK,), jnp.int32),
            [pltpu.VMEM((CHUNK, D), jnp.float32)] * 2,
            pltpu.VMEM((DEG_ROWS, 128), jnp.float32),
            pltpu.VMEM((128,), jnp.int32),
            pltpu.VMEM_SHARED((NP, D), jnp.float32),
            pltpu.VMEM_SHARED((DEG_ROWS, 128), jnp.float32),
            [pltpu.SemaphoreType.DMA] * 2,
            [pltpu.SemaphoreType.DMA] * 2,
        ],
        compiler_params=pltpu.CompilerParams(needs_layout_passes=False),
    )
    def seg_kernel(x_hbm, eidx_hbm, zeros_hbm, iota_hbm,
                   s_out_hbm, deg_out_hbm,
                   gidx, sidx, rows, deg_v, iota_v, acc, acc_deg, gsem, ssem):
        cid = lax.axis_index("c")
        sid = lax.axis_index("s")
        r0 = sid * ROWS_PER_TILE
        d0 = sid * (DEG_ROWS // NS)
        gbase = (cid * NS + sid) * EDGES_PER_TILE
        sbase = ((1 - cid) * NS + sid) * EDGES_PER_TILE
        pltpu.sync_copy(zeros_hbm.at[pl.ds(0, DEG_ROWS)], deg_v)
        pltpu.sync_copy(zeros_hbm.at[pl.ds(0, DEG_ROWS // NS)],
                        acc_deg.at[pl.ds(d0, DEG_ROWS // NS)])
        pltpu.sync_copy(iota_hbm, iota_v)
        pltpu.sync_copy(zeros_hbm, acc.at[pl.ds(r0, ROWS_PER_TILE)])
        plsc.subcore_barrier()

        ones16 = jnp.ones((16,), jnp.float32)

        def blk(kb, carry):
            pltpu.sync_copy(eidx_hbm.at[pl.ds(gbase + kb * IBLK, IBLK)],
                            gidx)
            pltpu.sync_copy(eidx_hbm.at[pl.ds(sbase + kb * IBLK, IBLK)],
                            sidx)
            gs = [gidx.at[pl.ds(j * CHUNK, CHUNK)] for j in range(CPB)]
            ss = [sidx.at[pl.ds(j * CHUNK, CHUNK)] for j in range(CPB)]
            pltpu.async_copy(x_hbm.at[gs[0]], rows[0], gsem[0])
            for j in range(CPB):
                b = j & 1
                nb = 1 - b
                if j < CPB - 1:
                    if j > 0:
                        pltpu.make_async_copy(rows[nb], acc.at[ss[j - 1]],
                                              ssem[nb]).wait()
                    pltpu.async_copy(x_hbm.at[gs[j + 1]], rows[nb], gsem[nb])
                pltpu.make_async_copy(x_hbm.at[gs[j]], rows[b],
                                      gsem[b]).wait()
                pltpu.async_copy(rows[b], acc.at[ss[j]], ssem[b], add=True)
                for l in range(CHUNK // 16):
                    idx = sidx[pl.ds(j * CHUNK + l * 16, 16)]
                    plsc.addupdate_scatter(
                        deg_v,
                        [lax.shift_right_logical(idx, 7),
                         lax.bitwise_and(idx, 127)],
                        ones16)
            pltpu.make_async_copy(rows[0], acc.at[ss[JE0]],
                                  ssem[0]).wait()
            pltpu.make_async_copy(rows[1], acc.at[ss[JE1]],
                                  ssem[1]).wait()
            return carry

        lax.fori_loop(0, NBLK, blk, 0)

        pltpu.sync_copy(deg_v, acc_deg.at[iota_v], add=True)
        plsc.subcore_barrier()
        pltpu.sync_copy(acc.at[pl.ds(r0, ROWS_PER_TILE)],
                        s_out_hbm.at[cid, pl.ds(r0, ROWS_PER_TILE)])
        pltpu.sync_copy(
            acc_deg.at[pl.ds(d0, DEG_ROWS // NS)],
            deg_out_hbm.at[cid, pl.ds(d0, DEG_ROWS // NS)],
        )

    return seg_kernel(x, eidx_flat, zeros_rows, iota128)


def _combine_body(x_ref, sa_ref, so_ref, da_ref, do_ref,
                  wst_ref, wbt_ref, wdt_ref, wa1_ref, wa2_ref,
                  bs_ref, bd_ref, ba1_ref, out_ref):
    xb = x_ref[...]
    xs = jnp.dot(xb, wst_ref[...], preferred_element_type=jnp.float32) + bs_ref[...]
    xbB = jnp.dot(xb, wbt_ref[...], preferred_element_type=jnp.float32) + bd_ref[...]
    wdt = wdt_ref[...]
    inc = jnp.dot(sa_ref[0], wdt, preferred_element_type=jnp.float32) + da_ref[...] * xbB
    outg = jnp.dot(so_ref[0], wdt, preferred_element_type=jnp.float32) + do_ref[...] * xbB
    wa1 = wa1_ref[...]
    wa2 = wa2_ref[...]
    ba1 = ba1_ref[...]

    def logit(r):
        h = jnp.tanh(jnp.dot(r, wa1, preferred_element_type=jnp.float32) + ba1)
        return jnp.sum(h * wa2, axis=1, keepdims=True)

    l0, l1, l2 = logit(xs), logit(inc), logit(outg)
    m = jnp.maximum(jnp.maximum(l0, l1), l2)
    e0 = jnp.exp(l0 - m)
    e1 = jnp.exp(l1 - m)
    e2 = jnp.exp(l2 - m)
    out_ref[...] = (e0 * xs + e1 * inc + e2 * outg) / (e0 + e1 + e2)


def _tc_combine(x, S, din, dout, WsT, WbT, WdT, Wa1T, wa2, bs, bd, ba1):
    grid = (N // BN,)
    wspec = lambda shape: pl.BlockSpec(shape, lambda i: tuple(0 for _ in shape))
    return pl.pallas_call(
        _combine_body,
        grid=grid,
        in_specs=[
            pl.BlockSpec((BN, D), lambda i: (i, 0)),
            pl.BlockSpec((1, BN, D), lambda i: (0, i, 0)),
            pl.BlockSpec((1, BN, D), lambda i: (1, i, 0)),
            pl.BlockSpec((BN, 1), lambda i: (i, 0)),
            pl.BlockSpec((BN, 1), lambda i: (i, 0)),
            wspec((D, D)),
            wspec((D, D)),
            wspec((D, D)),
            wspec((D, 16)),
            wspec((1, 16)),
            wspec((1, D)),
            wspec((1, D)),
            wspec((1, 16)),
        ],
        out_specs=pl.BlockSpec((BN, D), lambda i: (i, 0)),
        out_shape=jax.ShapeDtypeStruct((N, D), jnp.float32),
    )(x, S, S, din, dout, WsT, WbT, WdT, Wa1T, wa2, bs, bd, ba1)


def kernel(x, edge_index, W_self, b_self, W_disc, b_disc, W_att1, b_att1, W_att2):
    x = x.astype(jnp.float32)
    xp = jnp.concatenate([x, jnp.zeros((NP - N, D), jnp.float32)], axis=0)
    eidx = edge_index.astype(jnp.int32).reshape(2 * NS * EDGES_PER_TILE)
    zeros_rows = jnp.zeros((ROWS_PER_TILE, D), jnp.float32)
    iota128 = jnp.arange(128, dtype=jnp.int32)
    S, degH = _sc_segment_sums(xp, eidx, zeros_rows, iota128)
    deg = degH.reshape(NC, DEG_ROWS * 128)[:, :N]
    din = deg[0].reshape(N, 1)
    dout = deg[1].reshape(N, 1)

    Wa = W_disc[:, :D]
    Wb = W_disc[:, D:]
    WsT = W_self.T
    WbT = Wb.T
    WdT = (Wa - Wb).T
    Wa1T = W_att1.T
    wa2 = W_att2.reshape(1, 16)
    bs = b_self.reshape(1, D)
    bd = b_disc.reshape(1, D)
    ba1 = b_att1.reshape(1, 16)
    return _tc_combine(x, S, din, dout, WsT, WbT, WdT, Wa1T, wa2, bs, bd, ba1)


# speedup vs baseline: 5.6734x; 1.0331x over previous
import functools

import jax
import jax.numpy as jnp
from jax import lax
from jax.experimental import pallas as pl
from jax.experimental.pallas import tpu as pltpu
from jax.experimental.pallas import tpu_sc as plsc

N = 10000
NP = 10240
E = 320000
D = 128
NC = 2
NS = 16
ROWS_PER_TILE = NP // NS
EDGES_PER_TILE = E // NS
CHUNK = 80
NCHUNK = EDGES_PER_TILE // CHUNK
CPB = 50
IBLK = CHUNK * CPB
NBLK = EDGES_PER_TILE // IBLK
JE0 = CPB - 1 if (CPB - 1) % 2 == 0 else CPB - 2
JE1 = CPB - 1 if (CPB - 1) % 2 == 1 else CPB - 2
PAD_NODE = 10200
DEG_ROWS = 128

BN = 1000


def _sc_segment_sums(x, eidx_flat, zeros_rows, iota128):
    mesh = plsc.VectorSubcoreMesh(
        core_axis_name="c", subcore_axis_name="s", num_cores=NC, num_subcores=NS
    )

    @functools.partial(
        pl.kernel,
        out_type=(
            jax.ShapeDtypeStruct((NC, NP, D), jnp.float32),
            jax.ShapeDtypeStruct((NC, DEG_ROWS, 128), jnp.float32),
        ),
        mesh=mesh,
        scratch_types=[
            pltpu.VMEM((IBLK,), jnp.int32),
            pltpu.VMEM((IBLK,), jnp.int32),
            [pltpu.VMEM((CHUNK, D), jnp.float32)] * 2,
            pltpu.VMEM((DEG_ROWS, 128), jnp.float32),
            pltpu.VMEM((128,), jnp.int32),
            pltpu.VMEM_SHARED((NP, D), jnp.float32),
            pltpu.VMEM_SHARED((DEG_ROWS, 128), jnp.float32),
            [pltpu.SemaphoreType.DMA] * 2,
            [pltpu.SemaphoreType.DMA] * 2,
        ],
        compiler_params=pltpu.CompilerParams(needs_layout_passes=False),
    )
    def seg_kernel(x_hbm, eidx_hbm, zeros_hbm, iota_hbm,
                   s_out_hbm, deg_out_hbm,
                   gidx, sidx, rows, deg_v, iota_v, acc, acc_deg, gsem, ssem):
        cid = lax.axis_index("c")
        sid = lax.axis_index("s")
        r0 = sid * ROWS_PER_TILE
        d0 = sid * (DEG_ROWS // NS)
        gbase = (cid * NS + sid) * EDGES_PER_TILE
        sbase = ((1 - cid) * NS + sid) * EDGES_PER_TILE
        pltpu.sync_copy(zeros_hbm.at[pl.ds(0, DEG_ROWS)], deg_v)
        pltpu.sync_copy(zeros_hbm.at[pl.ds(0, DEG_ROWS // NS)],
                        acc_deg.at[pl.ds(d0, DEG_ROWS // NS)])
        pltpu.sync_copy(iota_hbm, iota_v)
        pltpu.sync_copy(zeros_hbm, acc.at[pl.ds(r0, ROWS_PER_TILE)])
        plsc.subcore_barrier()

        ones16 = jnp.ones((16,), jnp.float32)

        def blk(kb, carry):
            pltpu.sync_copy(eidx_hbm.at[pl.ds(gbase + kb * IBLK, IBLK)],
                            gidx)
            pltpu.sync_copy(eidx_hbm.at[pl.ds(sbase + kb * IBLK, IBLK)],
                            sidx)
            gs = [gidx.at[pl.ds(j * CHUNK, CHUNK)] for j in range(CPB)]
            ss = [sidx.at[pl.ds(j * CHUNK, CHUNK)] for j in range(CPB)]
            pltpu.async_copy(x_hbm.at[gs[0]], rows[0], gsem[0])
            for j in range(CPB):
                b = j & 1
                nb = 1 - b
                if j < CPB - 1:
                    if j > 0:
                        pltpu.make_async_copy(rows[nb], acc.at[ss[j - 1]],
                                              ssem[nb]).wait()
                    pltpu.async_copy(x_hbm.at[gs[j + 1]], rows[nb], gsem[nb])
                pltpu.make_async_copy(x_hbm.at[gs[j]], rows[b],
                                      gsem[b]).wait()
                pltpu.async_copy(rows[b], acc.at[ss[j]], ssem[b], add=True)
                for l in range(CHUNK // 16):
                    idx = sidx[pl.ds(j * CHUNK + l * 16, 16)]
                    plsc.addupdate_scatter(
                        deg_v,
                        [lax.shift_right_logical(idx, 7),
                         lax.bitwise_and(idx, 127)],
                        ones16)
            pltpu.make_async_copy(rows[0], acc.at[ss[JE0]],
                                  ssem[0]).wait()
            pltpu.make_async_copy(rows[1], acc.at[ss[JE1]],
                                  ssem[1]).wait()
            return carry

        lax.fori_loop(0, NBLK, blk, 0)

        pltpu.sync_copy(deg_v, acc_deg.at[iota_v], add=True)
        plsc.subcore_barrier()
        pltpu.sync_copy(acc.at[pl.ds(r0, ROWS_PER_TILE)],
                        s_out_hbm.at[cid, pl.ds(r0, ROWS_PER_TILE)])
        pltpu.sync_copy(
            acc_deg.at[pl.ds(d0, DEG_ROWS // NS)],
            deg_out_hbm.at[cid, pl.ds(d0, DEG_ROWS // NS)],
        )

    return seg_kernel(x, eidx_flat, zeros_rows, iota128)


def _combine_body(x_ref, sa_ref, so_ref, da_ref, do_ref,
                  wst_ref, wbt_ref, wdt_ref, wa1_ref, wa2_ref,
                  bs_ref, bd_ref, ba1_ref, out_ref):
    xb = x_ref[...]
    xs = jnp.dot(xb, wst_ref[...], preferred_element_type=jnp.float32) + bs_ref[...]
    xbB = jnp.dot(xb, wbt_ref[...], preferred_element_type=jnp.float32) + bd_ref[...]
    wdt = wdt_ref[...]
    inc = jnp.dot(sa_ref[0], wdt, preferred_element_type=jnp.float32) + da_ref[...] * xbB
    outg = jnp.dot(so_ref[0], wdt, preferred_element_type=jnp.float32) + do_ref[...] * xbB
    wa1 = wa1_ref[...]
    wa2 = wa2_ref[...]
    ba1 = ba1_ref[...]

    def logit(r):
        h = jnp.tanh(jnp.dot(r, wa1, preferred_element_type=jnp.float32) + ba1)
        return jnp.sum(h * wa2, axis=1, keepdims=True)

    l0, l1, l2 = logit(xs), logit(inc), logit(outg)
    m = jnp.maximum(jnp.maximum(l0, l1), l2)
    e0 = jnp.exp(l0 - m)
    e1 = jnp.exp(l1 - m)
    e2 = jnp.exp(l2 - m)
    out_ref[...] = (e0 * xs + e1 * inc + e2 * outg) / (e0 + e1 + e2)


def _tc_combine(x, S, din, dout, WsT, WbT, WdT, Wa1T, wa2, bs, bd, ba1):
    grid = (N // BN,)
    wspec = lambda shape: pl.BlockSpec(shape, lambda i: tuple(0 for _ in shape))
    return pl.pallas_call(
        _combine_body,
        grid=grid,
        in_specs=[
            pl.BlockSpec((BN, D), lambda i: (i, 0)),
            pl.BlockSpec((1, BN, D), lambda i: (0, i, 0)),
            pl.BlockSpec((1, BN, D), lambda i: (1, i, 0)),
            pl.BlockSpec((BN, 1), lambda i: (i, 0)),
            pl.BlockSpec((BN, 1), lambda i: (i, 0)),
            wspec((D, D)),
            wspec((D, D)),
            wspec((D, D)),
            wspec((D, 16)),
            wspec((1, 16)),
            wspec((1, D)),
            wspec((1, D)),
            wspec((1, 16)),
        ],
        out_specs=pl.BlockSpec((BN, D), lambda i: (i, 0)),
        out_shape=jax.ShapeDtypeStruct((N, D), jnp.float32),
    )(x, S, S, din, dout, WsT, WbT, WdT, Wa1T, wa2, bs, bd, ba1)


def kernel(x, edge_index, W_self, b_self, W_disc, b_disc, W_att1, b_att1, W_att2):
    x = x.astype(jnp.float32)
    xp = jnp.concatenate([x, jnp.zeros((NP - N, D), jnp.float32)], axis=0)
    eidx = edge_index.astype(jnp.int32).reshape(2 * NS * EDGES_PER_TILE)
    zeros_rows = jnp.zeros((ROWS_PER_TILE, D), jnp.float32)
    iota128 = jnp.arange(128, dtype=jnp.int32)
    S, degH = _sc_segment_sums(xp, eidx, zeros_rows, iota128)
    deg = degH.reshape(NC, DEG_ROWS * 128)[:, :N]
    din = deg[0].reshape(N, 1)
    dout = deg[1].reshape(N, 1)

    Wa = W_disc[:, :D]
    Wb = W_disc[:, D:]
    WsT = W_self.T
    WbT = Wb.T
    WdT = (Wa - Wb).T
    Wa1T = W_att1.T
    wa2 = W_att2.reshape(1, 16)
    bs = b_self.reshape(1, D)
    bd = b_disc.reshape(1, D)
    ba1 = b_att1.reshape(1, 16)
    return _tc_combine(x, S, din, dout, WsT, WbT, WdT, Wa1T, wa2, bs, bd, ba1)
